# baseline (speedup 1.0000x reference)
import functools

import jax
import jax.numpy as jnp
from jax import lax
from jax.experimental import pallas as pl
from jax.experimental.pallas import tpu as pltpu
from jax.experimental.pallas import tpu_sc as plsc

_N_DIMS = 10
_DELTA = 1.0 / 4095.0


def _round_to_bf16(x):
    u = lax.bitcast_convert_type(x, jnp.uint32)
    lsb = (u >> 16) & jnp.uint32(1)
    u2 = (u + jnp.uint32(0x7FFF) + lsb) & jnp.uint32(0xFFFF0000)
    return lax.bitcast_convert_type(u2, jnp.float32)


def _sc_row_tables(codebook, w_flat, rows):
    n_workers = 32
    rows_per_w = rows // n_workers
    tbl_per_w = rows_per_w * 16
    mesh = plsc.VectorSubcoreMesh(core_axis_name="c", subcore_axis_name="s")

    @functools.partial(
        pl.kernel,
        mesh=mesh,
        compiler_params=pltpu.CompilerParams(needs_layout_passes=False),
        out_type=(
            jax.ShapeDtypeStruct((rows * 16,), jnp.int32),
            jax.ShapeDtypeStruct((rows * 16,), jnp.float32),
        ),
        scratch_types=[
            pltpu.VMEM((1024,), jnp.float32),
            pltpu.VMEM((48,), jnp.float32),
            pltpu.VMEM((tbl_per_w,), jnp.int32),
            pltpu.VMEM((tbl_per_w,), jnp.float32),
        ],
    )
    def prep(cb_hbm, w_hbm, bp_hbm, vals_hbm, cb_v, w_v, bp_buf, vals_buf):
        wid = lax.axis_index("s") * 2 + lax.axis_index("c")
        pltpu.sync_copy(cb_hbm, cb_v)
        pltpu.sync_copy(w_hbm, w_v)
        tw0 = _round_to_bf16(w_v[pl.ds(0, 16)])
        tw1 = _round_to_bf16(w_v[pl.ds(16, 16)])
        tw2 = _round_to_bf16(w_v[pl.ds(32, 16)])
        lanes = lax.iota(jnp.int32, 16)
        pow2 = jnp.where(lanes < _N_DIMS, jnp.int32(1) << lanes, jnp.int32(0))
        delta = jnp.float32(_DELTA)
        row_base = wid * rows_per_w

        def row_body(rl, carry):
            r_f = (row_base + rl).astype(jnp.float32)
            grt = _round_to_bf16(jnp.full((16,), r_f * delta, jnp.float32))
            a = jnp.float32(0.5) * tw0 + grt * tw1
            pred0 = a > 0
            idx0 = jnp.sum(jnp.where(pred0, pow2, jnp.int32(0)))
            lo = jnp.zeros((16,), jnp.int32)
            hi = jnp.full((16,), 4096, jnp.int32)
            for _ in range(12):
                mid = (lo + hi) >> 1
                tgc = _round_to_bf16(mid.astype(jnp.float32) * delta)
                f = a + tgc * tw2
                flipped = (f > 0) != pred0
                hi = jnp.where(flipped, mid, hi)
                lo = jnp.where(flipped, lo, mid)
            sbp, spow = plsc.sort_key_val(hi, pow2)
            incl = plsc.cumsum(spow)
            seg_idx = idx0 ^ (incl - spow)
            vals = plsc.load_gather(cb_v, [seg_idx])
            bp_buf[pl.ds(rl * 16, 16)] = sbp
            vals_buf[pl.ds(rl * 16, 16)] = vals
            return carry

        lax.fori_loop(0, rows_per_w, row_body, 0)
        pltpu.sync_copy(bp_buf, bp_hbm.at[pl.ds(wid * tbl_per_w, tbl_per_w)])
        pltpu.sync_copy(vals_buf, vals_hbm.at[pl.ds(wid * tbl_per_w, tbl_per_w)])

    return prep(codebook, w_flat)


def _tc_expand_body(bp_ref, vals_ref, out_ref):
    br, cc = out_ref.shape
    rt, ct = 8, 2048
    for r0 in range(0, br, rt):
        for c0 in range(0, cc, ct):
            col = lax.broadcasted_iota(jnp.int32, (1, ct), 1) + c0
            acc = jnp.broadcast_to(vals_ref[r0 : r0 + rt, 0:1], (rt, ct))
            for k in range(_N_DIMS):
                acc = jnp.where(
                    col >= bp_ref[r0 : r0 + rt, k : k + 1],
                    vals_ref[r0 : r0 + rt, k + 1 : k + 2],
                    acc,
                )
            out_ref[r0 : r0 + rt, c0 : c0 + ct] = acc


def _tc_expand(bp, vals, rows, cols, block_rows):
    return pl.pallas_call(
        _tc_expand_body,
        grid=(rows // block_rows,),
        in_specs=[
            pl.BlockSpec((block_rows, 16), lambda i: (i, 0)),
            pl.BlockSpec((block_rows, 16), lambda i: (i, 0)),
        ],
        out_specs=pl.BlockSpec((block_rows, cols), lambda i: (i, 0)),
        out_shape=jax.ShapeDtypeStruct((rows, cols), jnp.float32),
    )(bp, vals)


def kernel(weight_matrix, codebook, W_proj):
    rows, cols = weight_matrix.shape
    w_flat = jnp.concatenate(
        [jnp.pad(W_proj[:, k], (0, 16 - _N_DIMS)) for k in range(3)]
    )
    bp, vals = _sc_row_tables(codebook, w_flat, rows)
    return _tc_expand(
        bp.reshape(rows, 16), vals.reshape(rows, 16), rows, cols, block_rows=512
    )


# speedup vs baseline: 1.0089x; 1.0089x over previous
import functools

import jax
import jax.numpy as jnp
from jax import lax
from jax.experimental import pallas as pl
from jax.experimental.pallas import tpu as pltpu
from jax.experimental.pallas import tpu_sc as plsc

_N_DIMS = 10
_DELTA = 1.0 / 4095.0


def _round_to_bf16(x):
    u = lax.bitcast_convert_type(x, jnp.uint32)
    lsb = (u >> 16) & jnp.uint32(1)
    u2 = (u + jnp.uint32(0x7FFF) + lsb) & jnp.uint32(0xFFFF0000)
    return lax.bitcast_convert_type(u2, jnp.float32)


def _sc_row_tables(codebook, w_flat, rows):
    n_workers = 32
    rows_per_w = rows // n_workers
    tbl_per_w = rows_per_w * 16
    mesh = plsc.VectorSubcoreMesh(core_axis_name="c", subcore_axis_name="s")

    @functools.partial(
        pl.kernel,
        mesh=mesh,
        compiler_params=pltpu.CompilerParams(needs_layout_passes=False),
        out_type=(
            jax.ShapeDtypeStruct((rows * 16,), jnp.int32),
            jax.ShapeDtypeStruct((rows * 16,), jnp.float32),
        ),
        scratch_types=[
            pltpu.VMEM((1024,), jnp.float32),
            pltpu.VMEM((48,), jnp.float32),
            pltpu.VMEM((tbl_per_w,), jnp.int32),
            pltpu.VMEM((tbl_per_w,), jnp.float32),
        ],
    )
    def prep(cb_hbm, w_hbm, bp_hbm, vals_hbm, cb_v, w_v, bp_buf, vals_buf):
        wid = lax.axis_index("s") * 2 + lax.axis_index("c")
        pltpu.sync_copy(cb_hbm, cb_v)
        pltpu.sync_copy(w_hbm, w_v)
        tw0 = _round_to_bf16(w_v[pl.ds(0, 16)])
        tw1 = _round_to_bf16(w_v[pl.ds(16, 16)])
        tw2 = _round_to_bf16(w_v[pl.ds(32, 16)])
        lanes = lax.iota(jnp.int32, 16)
        pow2 = jnp.where(lanes < _N_DIMS, jnp.int32(1) << lanes, jnp.int32(0))
        delta = jnp.float32(_DELTA)
        row_base = wid * rows_per_w

        def row_body(rl, carry):
            r_f = (row_base + rl).astype(jnp.float32)
            grt = _round_to_bf16(jnp.full((16,), r_f * delta, jnp.float32))
            a = jnp.float32(0.5) * tw0 + grt * tw1
            pred0 = a > 0
            idx0 = jnp.sum(jnp.where(pred0, pow2, jnp.int32(0)))
            lo = jnp.zeros((16,), jnp.int32)
            hi = jnp.full((16,), 4096, jnp.int32)
            for _ in range(12):
                mid = (lo + hi) >> 1
                tgc = _round_to_bf16(mid.astype(jnp.float32) * delta)
                f = a + tgc * tw2
                flipped = (f > 0) != pred0
                hi = jnp.where(flipped, mid, hi)
                lo = jnp.where(flipped, lo, mid)
            sbp, spow = plsc.sort_key_val(hi, pow2)
            incl = plsc.cumsum(spow)
            seg_idx = idx0 ^ (incl - spow)
            vals = plsc.load_gather(cb_v, [seg_idx])
            bp_buf[pl.ds(rl * 16, 16)] = sbp
            vals_buf[pl.ds(rl * 16, 16)] = vals
            return carry

        lax.fori_loop(0, rows_per_w, row_body, 0)
        pltpu.sync_copy(bp_buf, bp_hbm.at[pl.ds(wid * tbl_per_w, tbl_per_w)])
        pltpu.sync_copy(vals_buf, vals_hbm.at[pl.ds(wid * tbl_per_w, tbl_per_w)])

    return prep(codebook, w_flat)


def _tc_expand_body(bp_ref, vals_ref, out_ref):
    br, cc = out_ref.shape
    rt, ct = 8, 2048
    for r0 in range(0, br, rt):
        for c0 in range(0, cc, ct):
            col = lax.broadcasted_iota(jnp.int32, (1, ct), 1) + c0
            acc = jnp.broadcast_to(vals_ref[r0 : r0 + rt, 0:1], (rt, ct))
            for k in range(_N_DIMS):
                acc = jnp.where(
                    col >= bp_ref[r0 : r0 + rt, k : k + 1],
                    vals_ref[r0 : r0 + rt, k + 1 : k + 2],
                    acc,
                )
            out_ref[r0 : r0 + rt, c0 : c0 + ct] = acc


def _tc_expand(bp, vals, rows, cols, block_rows):
    return pl.pallas_call(
        _tc_expand_body,
        grid=(rows // block_rows,),
        in_specs=[
            pl.BlockSpec((block_rows, 16), lambda i: (i, 0)),
            pl.BlockSpec((block_rows, 16), lambda i: (i, 0)),
        ],
        out_specs=pl.BlockSpec((block_rows, cols), lambda i: (i, 0)),
        out_shape=jax.ShapeDtypeStruct((rows, cols), jnp.float32),
    )(bp, vals)


def kernel(weight_matrix, codebook, W_proj):
    rows, cols = weight_matrix.shape
    w_flat = jnp.concatenate(
        [jnp.pad(W_proj[:, k], (0, 16 - _N_DIMS)) for k in range(3)]
    )
    bp, vals = _sc_row_tables(codebook, w_flat, rows)
    return _tc_expand(
        bp.reshape(rows, 16), vals.reshape(rows, 16), rows, cols, block_rows=128
    )


# speedup vs baseline: 1.0151x; 1.0062x over previous
import functools

import jax
import jax.numpy as jnp
from jax import lax
from jax.experimental import pallas as pl
from jax.experimental.pallas import tpu as pltpu
from jax.experimental.pallas import tpu_sc as plsc

_N_DIMS = 10
_DELTA = 1.0 / 4095.0


def _round_to_bf16(x):
    u = lax.bitcast_convert_type(x, jnp.uint32)
    lsb = (u >> 16) & jnp.uint32(1)
    u2 = (u + jnp.uint32(0x7FFF) + lsb) & jnp.uint32(0xFFFF0000)
    return lax.bitcast_convert_type(u2, jnp.float32)


def _sc_row_tables(codebook, w_flat, rows):
    n_workers = 32
    rows_per_w = rows // n_workers
    tbl_per_w = rows_per_w * 16
    mesh = plsc.VectorSubcoreMesh(core_axis_name="c", subcore_axis_name="s")

    @functools.partial(
        pl.kernel,
        mesh=mesh,
        compiler_params=pltpu.CompilerParams(needs_layout_passes=False),
        out_type=(
            jax.ShapeDtypeStruct((rows * 16,), jnp.int32),
            jax.ShapeDtypeStruct((rows * 16,), jnp.float32),
        ),
        scratch_types=[
            pltpu.VMEM((1024,), jnp.float32),
            pltpu.VMEM((48,), jnp.float32),
            pltpu.VMEM((tbl_per_w,), jnp.int32),
            pltpu.VMEM((tbl_per_w,), jnp.float32),
        ],
    )
    def prep(cb_hbm, w_hbm, bp_hbm, vals_hbm, cb_v, w_v, bp_buf, vals_buf):
        wid = lax.axis_index("s") * 2 + lax.axis_index("c")
        pltpu.sync_copy(cb_hbm, cb_v)
        pltpu.sync_copy(w_hbm, w_v)
        tw0 = _round_to_bf16(w_v[pl.ds(0, 16)])
        tw1 = _round_to_bf16(w_v[pl.ds(16, 16)])
        tw2 = _round_to_bf16(w_v[pl.ds(32, 16)])
        lanes = lax.iota(jnp.int32, 16)
        pow2 = jnp.where(lanes < _N_DIMS, jnp.int32(1) << lanes, jnp.int32(0))
        delta = jnp.float32(_DELTA)
        row_base = wid * rows_per_w

        def row_body(rl, carry):
            r_f = (row_base + rl).astype(jnp.float32)
            grt = _round_to_bf16(jnp.full((16,), r_f * delta, jnp.float32))
            a = jnp.float32(0.5) * tw0 + grt * tw1
            pred0 = a > 0
            idx0 = jnp.sum(jnp.where(pred0, pow2, jnp.int32(0)))
            lo = jnp.zeros((16,), jnp.int32)
            hi = jnp.full((16,), 4096, jnp.int32)
            for _ in range(12):
                mid = (lo + hi) >> 1
                tgc = _round_to_bf16(mid.astype(jnp.float32) * delta)
                f = a + tgc * tw2
                flipped = (f > 0) != pred0
                hi = jnp.where(flipped, mid, hi)
                lo = jnp.where(flipped, lo, mid)
            sbp, spow = plsc.sort_key_val(hi, pow2)
            incl = plsc.cumsum(spow)
            seg_idx = idx0 ^ (incl - spow)
            vals = plsc.load_gather(cb_v, [seg_idx])
            bp_buf[pl.ds(rl * 16, 16)] = sbp
            vals_buf[pl.ds(rl * 16, 16)] = vals
            return carry

        lax.fori_loop(0, rows_per_w, row_body, 0)
        pltpu.sync_copy(bp_buf, bp_hbm.at[pl.ds(wid * tbl_per_w, tbl_per_w)])
        pltpu.sync_copy(vals_buf, vals_hbm.at[pl.ds(wid * tbl_per_w, tbl_per_w)])

    return prep(codebook, w_flat)


def _tc_expand_body(bp_ref, vals_ref, out_ref):
    br, cc = out_ref.shape
    rt, ct = 8, 4096
    for r0 in range(0, br, rt):
        for c0 in range(0, cc, ct):
            col = lax.broadcasted_iota(jnp.int32, (1, ct), 1) + c0
            acc = jnp.broadcast_to(vals_ref[r0 : r0 + rt, 0:1], (rt, ct))
            for k in range(_N_DIMS):
                acc = jnp.where(
                    col >= bp_ref[r0 : r0 + rt, k : k + 1],
                    vals_ref[r0 : r0 + rt, k + 1 : k + 2],
                    acc,
                )
            out_ref[r0 : r0 + rt, c0 : c0 + ct] = acc


def _tc_expand(bp, vals, rows, cols, block_rows):
    return pl.pallas_call(
        _tc_expand_body,
        grid=(rows // block_rows,),
        in_specs=[
            pl.BlockSpec((block_rows, 16), lambda i: (i, 0)),
            pl.BlockSpec((block_rows, 16), lambda i: (i, 0)),
        ],
        out_specs=pl.BlockSpec((block_rows, cols), lambda i: (i, 0)),
        out_shape=jax.ShapeDtypeStruct((rows, cols), jnp.float32),
    )(bp, vals)


def kernel(weight_matrix, codebook, W_proj):
    rows, cols = weight_matrix.shape
    w_flat = jnp.concatenate(
        [jnp.pad(W_proj[:, k], (0, 16 - _N_DIMS)) for k in range(3)]
    )
    bp, vals = _sc_row_tables(codebook, w_flat, rows)
    return _tc_expand(
        bp.reshape(rows, 16), vals.reshape(rows, 16), rows, cols, block_rows=128
    )
